# baseline (speedup 1.0000x reference)
import functools

import jax
import jax.numpy as jnp
from jax import lax
from jax.experimental import pallas as pl
from jax.experimental.pallas import tpu as pltpu
from jax.experimental.pallas import tpu_sc as plsc

N = 10000
E = 320000
D = 128
H = 128
C = 64

NC = 2
NS = 16
NW = NC * NS
EPW = E // NW
CHUNK = 96
NCHUNKS = 105
EPP = NCHUNKS * CHUNK
PADN = 10240
DEAD = N
RPT = PADN // NS


def _make_agg(h):
    mesh = plsc.VectorSubcoreMesh(core_axis_name="c", subcore_axis_name="s")

    @functools.partial(
        pl.kernel,
        out_type=jax.ShapeDtypeStruct((NC, PADN, h), jnp.float32),
        mesh=mesh,
        scratch_types=[
            pltpu.VMEM((EPP,), jnp.int32),
            pltpu.VMEM((NCHUNKS, CHUNK), jnp.int32),
            pltpu.VMEM((CHUNK, h), jnp.float32),
            pltpu.VMEM((CHUNK, h), jnp.float32),
            pltpu.VMEM_SHARED((PADN, h), jnp.float32),
            pltpu.SemaphoreType.DMA,
            pltpu.SemaphoreType.DMA,
            pltpu.SemaphoreType.DMA,
            pltpu.SemaphoreType.DMA,
        ],
    )
    def agg(srcr_hbm, dstr_hbm, y_hbm, out_hbm, src_v, dst_v, buf_a, buf_b,
            acc_sh, sem_is, sem_id, sem_a, sem_b):
        cid = lax.axis_index("c")
        sid = lax.axis_index("s")
        wid = cid * NS + sid

        cp_s = pltpu.async_copy(srcr_hbm.at[wid], src_v, sem_is)
        cp_d = pltpu.async_copy(dstr_hbm.at[wid], dst_v, sem_id)

        zeros = jnp.zeros((16,), jnp.float32)

        @pl.loop(0, CHUNK)
        def _zero_row(r):
            for cc in range(h // 16):
                buf_a[r, pl.ds(cc * 16, 16)] = zeros

        row0 = sid * RPT
        full, rem = divmod(RPT, CHUNK)
        for k in range(full):
            pltpu.sync_copy(buf_a,
                            acc_sh.at[pl.ds(row0 + k * CHUNK, CHUNK)])
        if rem:
            pltpu.sync_copy(buf_a.at[pl.ds(0, rem)],
                            acc_sh.at[pl.ds(row0 + full * CHUNK, rem)])
        cp_s.wait()
        cp_d.wait()
        plsc.subcore_barrier()

        def sidx(t):
            return src_v.at[pl.ds(t * CHUNK, CHUNK)]

        pltpu.async_copy(y_hbm.at[sidx(0)], buf_a, sem_a)

        @pl.loop(0, NCHUNKS - 1, step=2)
        def _pipe(t):
            pltpu.async_copy(y_hbm.at[sidx(t + 1)], buf_b, sem_b)
            pltpu.make_async_copy(y_hbm.at[sidx(t)], buf_a, sem_a).wait()
            pltpu.sync_copy(buf_a, acc_sh.at[dst_v.at[t]], add=True)

            @pl.when(t + 2 < NCHUNKS)
            def _():
                pltpu.async_copy(y_hbm.at[sidx(t + 2)], buf_a, sem_a)

            pltpu.make_async_copy(y_hbm.at[sidx(t + 1)], buf_b, sem_b).wait()
            pltpu.sync_copy(buf_b, acc_sh.at[dst_v.at[t + 1]], add=True)

        if NCHUNKS % 2:
            pltpu.make_async_copy(
                y_hbm.at[sidx(NCHUNKS - 1)], buf_a, sem_a).wait()
            pltpu.sync_copy(buf_a, acc_sh.at[dst_v.at[NCHUNKS - 1]], add=True)

        plsc.subcore_barrier()

        pltpu.sync_copy(acc_sh.at[pl.ds(row0, RPT)],
                        out_hbm.at[cid, pl.ds(row0, RPT)])

    return agg


_agg128 = _make_agg(H)


BM = 2000


def _first_body(x_ref, w_ref, n_ref, o_ref):
    o_ref[...] = jnp.dot(x_ref[...], w_ref[...],
                         preferred_element_type=jnp.float32) * n_ref[...]


def _first(x, w, norm):
    return pl.pallas_call(
        _first_body,
        grid=(N // BM,),
        in_specs=[
            pl.BlockSpec((BM, D), lambda i: (i, 0)),
            pl.BlockSpec((D, H), lambda i: (0, 0)),
            pl.BlockSpec((BM, 1), lambda i: (i, 0)),
        ],
        out_specs=pl.BlockSpec((BM, H), lambda i: (i, 0)),
        out_shape=jax.ShapeDtypeStruct((N, H), jnp.float32),
    )(x, w, norm)


def _comb_body(z_ref, n_ref, b_ref, w_ref, o_ref):
    hcur = (z_ref[0] + z_ref[1]) * n_ref[...] + b_ref[...]
    o_ref[...] = jnp.dot(hcur, w_ref[...],
                         preferred_element_type=jnp.float32) * n_ref[...]


def _comb(z, norm, b, w, wout):
    hin = z.shape[-1]
    return pl.pallas_call(
        _comb_body,
        grid=(N // BM,),
        in_specs=[
            pl.BlockSpec((NC, BM, hin), lambda i: (0, i, 0)),
            pl.BlockSpec((BM, 1), lambda i: (i, 0)),
            pl.BlockSpec((1, hin), lambda i: (0, 0)),
            pl.BlockSpec((hin, wout), lambda i: (0, 0)),
        ],
        out_specs=pl.BlockSpec((BM, wout), lambda i: (i, 0)),
        out_shape=jax.ShapeDtypeStruct((N, wout), jnp.float32),
    )(z, norm, b.reshape(1, hin), w)


def _scale_body(z_ref, n_ref, b_ref, o_ref):
    o_ref[...] = ((z_ref[0] + z_ref[1]) * n_ref[...] + b_ref[...]) * n_ref[...]


def _scale(z, norm, b):
    hin = z.shape[-1]
    return pl.pallas_call(
        _scale_body,
        grid=(N // BM,),
        in_specs=[
            pl.BlockSpec((NC, BM, hin), lambda i: (0, i, 0)),
            pl.BlockSpec((BM, 1), lambda i: (i, 0)),
            pl.BlockSpec((1, hin), lambda i: (0, 0)),
        ],
        out_specs=pl.BlockSpec((BM, hin), lambda i: (i, 0)),
        out_shape=jax.ShapeDtypeStruct((N, hin), jnp.float32),
    )(z, norm, b.reshape(1, hin))


def _finmm_body(z_ref, w_ref, n_ref, b_ref, o_ref):
    zsum = z_ref[0] + z_ref[1]
    o_ref[...] = jnp.dot(zsum, w_ref[...],
                         preferred_element_type=jnp.float32) * n_ref[...] \
        + b_ref[...]


def _finmm(z, w, norm, b):
    hin = z.shape[-1]
    wout = w.shape[-1]
    return pl.pallas_call(
        _finmm_body,
        grid=(N // BM,),
        in_specs=[
            pl.BlockSpec((NC, BM, hin), lambda i: (0, i, 0)),
            pl.BlockSpec((hin, wout), lambda i: (0, 0)),
            pl.BlockSpec((BM, 1), lambda i: (i, 0)),
            pl.BlockSpec((1, wout), lambda i: (0, 0)),
        ],
        out_specs=pl.BlockSpec((BM, wout), lambda i: (i, 0)),
        out_shape=jax.ShapeDtypeStruct((N, wout), jnp.float32),
    )(z, w, norm, b.reshape(1, wout))


def kernel(features, edge_index, norm, W0, b0, W1, b1, W2, b2):
    pad = ((0, 0), (0, EPP - EPW))
    srcr = jnp.pad(edge_index[0].reshape(NW, EPW), pad)
    deadpad = jnp.broadcast_to(DEAD + jnp.arange(NW, dtype=jnp.int32)[:, None],
                               (NW, EPP - EPW))
    dstr = jnp.concatenate(
        [edge_index[1].reshape(NW, EPW), deadpad],
        axis=1).reshape(NW, NCHUNKS, CHUNK)
    y0 = _first(features, W0, norm)
    z0 = _agg128(srcr, dstr, y0)
    y1 = _comb(z0, norm, b0, W1, H)
    z1 = _agg128(srcr, dstr, y1)
    y2 = _scale(z1, norm, b1)
    z2 = _agg128(srcr, dstr, y2)
    h3 = _finmm(z2, W2, norm, b2)
    return h3


# speedup vs baseline: 1.6342x; 1.6342x over previous
import functools

import jax
import jax.numpy as jnp
from jax import lax
from jax.experimental import pallas as pl
from jax.experimental.pallas import tpu as pltpu
from jax.experimental.pallas import tpu_sc as plsc

N = 10000
E = 320000
D = 128
H = 128
C = 64

NC = 2
NS = 16
NW = NC * NS
EPW = E // NW
CHUNK = 80
NCHUNKS = EPW // CHUNK
PADN = 10240
RPT = PADN // NS
ZR = 80
NZ = RPT // ZR


def _make_agg(h):
    mesh = plsc.VectorSubcoreMesh(core_axis_name="c", subcore_axis_name="s")

    @functools.partial(
        pl.kernel,
        out_type=jax.ShapeDtypeStruct((NC, PADN, h), jnp.float32),
        mesh=mesh,
        scratch_types=[
            pltpu.VMEM((EPW,), jnp.int32),
            pltpu.VMEM((NCHUNKS, CHUNK), jnp.int32),
            pltpu.VMEM((CHUNK, h), jnp.float32),
            pltpu.VMEM((CHUNK, h), jnp.float32),
            pltpu.VMEM_SHARED((PADN, h), jnp.float32),
            pltpu.SemaphoreType.DMA,
            pltpu.SemaphoreType.DMA,
            pltpu.SemaphoreType.DMA,
            pltpu.SemaphoreType.DMA,
        ],
    )
    def agg(srcr_hbm, dstr_hbm, y_hbm, out_hbm, src_v, dst_v, buf_a, buf_b,
            acc_sh, sem_is, sem_id, sem_a, sem_b):
        cid = lax.axis_index("c")
        sid = lax.axis_index("s")
        wid = cid * NS + sid

        cp_s = pltpu.async_copy(srcr_hbm.at[wid], src_v, sem_is)
        cp_d = pltpu.async_copy(dstr_hbm.at[wid], dst_v, sem_id)

        zeros = jnp.zeros((16,), jnp.float32)

        @pl.loop(0, ZR)
        def _zero_row(r):
            for cc in range(h // 16):
                buf_a[r, pl.ds(cc * 16, 16)] = zeros

        row0 = sid * RPT
        for k in range(NZ):
            pltpu.sync_copy(buf_a.at[pl.ds(0, ZR)],
                            acc_sh.at[pl.ds(row0 + k * ZR, ZR)])
        cp_s.wait()
        cp_d.wait()
        plsc.subcore_barrier()

        def sidx(t):
            return src_v.at[pl.ds(t * CHUNK, CHUNK)]

        pltpu.async_copy(y_hbm.at[sidx(0)], buf_a, sem_a)

        @pl.loop(0, NCHUNKS - 1, step=2)
        def _pipe(t):
            pltpu.async_copy(y_hbm.at[sidx(t + 1)], buf_b, sem_b)
            pltpu.make_async_copy(y_hbm.at[sidx(t)], buf_a, sem_a).wait()
            pltpu.sync_copy(buf_a, acc_sh.at[dst_v.at[t]], add=True)

            @pl.when(t + 2 < NCHUNKS)
            def _():
                pltpu.async_copy(y_hbm.at[sidx(t + 2)], buf_a, sem_a)

            pltpu.make_async_copy(y_hbm.at[sidx(t + 1)], buf_b, sem_b).wait()
            pltpu.sync_copy(buf_b, acc_sh.at[dst_v.at[t + 1]], add=True)

        if NCHUNKS % 2:
            pltpu.make_async_copy(
                y_hbm.at[sidx(NCHUNKS - 1)], buf_a, sem_a).wait()
            pltpu.sync_copy(buf_a, acc_sh.at[dst_v.at[NCHUNKS - 1]], add=True)

        plsc.subcore_barrier()

        for k in range(NZ):
            r = row0 + k * ZR
            pltpu.sync_copy(acc_sh.at[pl.ds(r, ZR)],
                            out_hbm.at[cid, pl.ds(r, ZR)])

    return agg


_agg128 = _make_agg(H)


BM = 2000


def _first_body(x_ref, w_ref, n_ref, o_ref):
    o_ref[...] = jnp.dot(x_ref[...], w_ref[...],
                         preferred_element_type=jnp.float32) * n_ref[...]


def _first(x, w, norm):
    return pl.pallas_call(
        _first_body,
        grid=(N // BM,),
        in_specs=[
            pl.BlockSpec((BM, D), lambda i: (i, 0)),
            pl.BlockSpec((D, H), lambda i: (0, 0)),
            pl.BlockSpec((BM, 1), lambda i: (i, 0)),
        ],
        out_specs=pl.BlockSpec((BM, H), lambda i: (i, 0)),
        out_shape=jax.ShapeDtypeStruct((N, H), jnp.float32),
    )(x, w, norm)


def _comb_body(z_ref, n_ref, b_ref, w_ref, o_ref):
    hcur = (z_ref[0] + z_ref[1]) * n_ref[...] + b_ref[...]
    o_ref[...] = jnp.dot(hcur, w_ref[...],
                         preferred_element_type=jnp.float32) * n_ref[...]


def _comb(z, norm, b, w, wout):
    hin = z.shape[-1]
    return pl.pallas_call(
        _comb_body,
        grid=(N // BM,),
        in_specs=[
            pl.BlockSpec((NC, BM, hin), lambda i: (0, i, 0)),
            pl.BlockSpec((BM, 1), lambda i: (i, 0)),
            pl.BlockSpec((1, hin), lambda i: (0, 0)),
            pl.BlockSpec((hin, wout), lambda i: (0, 0)),
        ],
        out_specs=pl.BlockSpec((BM, wout), lambda i: (i, 0)),
        out_shape=jax.ShapeDtypeStruct((N, wout), jnp.float32),
    )(z, norm, b.reshape(1, hin), w)


def _scale_body(z_ref, n_ref, b_ref, o_ref):
    o_ref[...] = ((z_ref[0] + z_ref[1]) * n_ref[...] + b_ref[...]) * n_ref[...]


def _scale(z, norm, b):
    hin = z.shape[-1]
    return pl.pallas_call(
        _scale_body,
        grid=(N // BM,),
        in_specs=[
            pl.BlockSpec((NC, BM, hin), lambda i: (0, i, 0)),
            pl.BlockSpec((BM, 1), lambda i: (i, 0)),
            pl.BlockSpec((1, hin), lambda i: (0, 0)),
        ],
        out_specs=pl.BlockSpec((BM, hin), lambda i: (i, 0)),
        out_shape=jax.ShapeDtypeStruct((N, hin), jnp.float32),
    )(z, norm, b.reshape(1, hin))


def _finmm_body(z_ref, w_ref, n_ref, b_ref, o_ref):
    zsum = z_ref[0] + z_ref[1]
    o_ref[...] = jnp.dot(zsum, w_ref[...],
                         preferred_element_type=jnp.float32) * n_ref[...] \
        + b_ref[...]


def _finmm(z, w, norm, b):
    hin = z.shape[-1]
    wout = w.shape[-1]
    return pl.pallas_call(
        _finmm_body,
        grid=(N // BM,),
        in_specs=[
            pl.BlockSpec((NC, BM, hin), lambda i: (0, i, 0)),
            pl.BlockSpec((hin, wout), lambda i: (0, 0)),
            pl.BlockSpec((BM, 1), lambda i: (i, 0)),
            pl.BlockSpec((1, wout), lambda i: (0, 0)),
        ],
        out_specs=pl.BlockSpec((BM, wout), lambda i: (i, 0)),
        out_shape=jax.ShapeDtypeStruct((N, wout), jnp.float32),
    )(z, w, norm, b.reshape(1, wout))


def kernel(features, edge_index, norm, W0, b0, W1, b1, W2, b2):
    srcr = edge_index[0].reshape(NW, EPW)
    dstr = edge_index[1].reshape(NW, NCHUNKS, CHUNK)
    y0 = _first(features, W0, norm)
    z0 = _agg128(srcr, dstr, y0)
    y1 = _comb(z0, norm, b0, W1, H)
    z1 = _agg128(srcr, dstr, y1)
    y2 = _scale(z1, norm, b1)
    z2 = _agg128(srcr, dstr, y2)
    h3 = _finmm(z2, W2, norm, b2)
    return h3
